# baseline (speedup 1.0000x reference)
import jax
import jax.numpy as jnp
from jax import lax
from jax.experimental import pallas as pl
from jax.experimental.pallas import tpu as pltpu
from jax.experimental.pallas import tpu_sc as plsc

B = 32
V = 1_000_000
BLK = 8192
CHUNK = 1024
VM = 737_280
S = V - VM
SC_NCH = 4
SC_CH = S // SC_NCH
SC_CH_VREGS = SC_CH // 16

_ROTS_EVEN = (13, 15, 26, 6)
_ROTS_ODD = (17, 29, 16, 24)
_K0 = 0
_K1 = 42
_K2 = _K0 ^ _K1 ^ 0x1BD11BDA
_KS = (_K0, _K1, _K2)
_TINY = float(jnp.finfo(jnp.float32).tiny)
_NEG_INF = float("-inf")


def _rotl(x, r):
    return (x << jnp.uint32(r)) + (x >> jnp.uint32(32 - r))


def _rotl_i32(x, r):
    return (x << jnp.int32(r)) + lax.shift_right_logical(x, jnp.int32(32 - r))


def _threefry_bits(x1, rotl, const):
    x0 = x1
    x1 = rotl(x1, _ROTS_EVEN[0]) ^ x0
    for r in _ROTS_EVEN[1:]:
        x0 = x0 + x1
        x1 = rotl(x1, r) ^ x0
    x0 = x0 + const(_KS[1])
    x1 = x1 + const(_KS[2] + 1)
    for g in range(1, 5):
        rots = _ROTS_EVEN if g % 2 == 0 else _ROTS_ODD
        for r in rots:
            x0 = x0 + x1
            x1 = rotl(x1, r) ^ x0
        x0 = x0 + const((_KS[(g + 1) % 3]) & 0xFFFFFFFF)
        x1 = x1 + const((_KS[(g + 2) % 3] + (g + 1)) & 0xFFFFFFFF)
    return x0 ^ x1


def _u32c(v):
    return jnp.uint32(v & 0xFFFFFFFF)


def _i32c(v):
    v &= 0xFFFFFFFF
    return jnp.int32(v - (1 << 32) if v >= (1 << 31) else v)


def _gumbel_from_bits_u32(bits):
    tiny = jnp.float32(_TINY)
    fb = (bits >> jnp.uint32(9)) | jnp.uint32(0x3F800000)
    floats = lax.bitcast_convert_type(fb, jnp.float32) - jnp.float32(1.0)
    u = jnp.maximum(tiny, floats + tiny)
    return -jnp.log(-jnp.log(u))


def _gumbel_from_bits_i32(bits):
    tiny = jnp.float32(_TINY)
    fb = lax.shift_right_logical(bits, jnp.int32(9)) | jnp.int32(0x3F800000)
    floats = lax.bitcast_convert_type(fb, jnp.float32) - jnp.float32(1.0)
    u = jnp.maximum(tiny, floats + tiny)
    return -jnp.log(-jnp.log(u))


def _sc_bits_body(iota_hbm, out_hbm, jv_ref, buf_ref):
    c = lax.axis_index("c")
    s = lax.axis_index("s")
    wid = s * 2 + c
    pltpu.sync_copy(iota_hbm, jv_ref)
    j0 = jv_ref[...] + (wid * V + VM + _K1)

    for ch in range(SC_NCH):
        @pl.loop(0, SC_CH_VREGS, init_carry=j0, unroll=4)
        def _body(t, j):
            bits = _threefry_bits(j, _rotl_i32, _i32c)
            buf_ref[pl.ds(t * 16, 16)] = bits
            return j + 16

        j0 = _body
        pltpu.sync_copy(buf_ref, out_hbm.at[pl.ds(wid * S + ch * SC_CH, SC_CH)])


def _sc_bits(iota16):
    return pl.kernel(
        _sc_bits_body,
        out_type=jax.ShapeDtypeStruct((B * S,), jnp.int32),
        mesh=plsc.VectorSubcoreMesh(core_axis_name="c", subcore_axis_name="s"),
        scratch_types=[
            pltpu.VMEM((16,), jnp.int32),
            pltpu.VMEM((SC_CH,), jnp.int32),
        ],
    )(iota16)


def _main_kernel(logits_ref, maxv_ref, idx_ref, accv_ref, accj_ref, base_ref):
    i = pl.program_id(0)
    nsteps = pl.num_programs(0)
    neg_inf = jnp.float32(_NEG_INF)

    @pl.when(i == 0)
    def _init():
        accv_ref[...] = jnp.full((B, CHUNK), neg_inf, jnp.float32)
        accj_ref[...] = jnp.zeros((B, CHUNK), jnp.uint32)
        row = lax.broadcasted_iota(jnp.int32, (B, CHUNK), 0)
        chunk_col = lax.broadcasted_iota(jnp.int32, (B, CHUNK), 1)
        base_ref[...] = (row * V + chunk_col + _K1).astype(jnp.uint32)

    c0 = i * BLK
    base = base_ref[...]
    accv = accv_ref[...]
    accj = accj_ref[...]
    for k in range(BLK // CHUNK):
        off = c0 + k * CHUNK
        jp = base + off.astype(jnp.uint32)
        v = _gumbel_from_bits_u32(_threefry_bits(jp, _rotl, _u32c)) + \
            logits_ref[:, k * CHUNK:(k + 1) * CHUNK]
        better = v > accv
        accv = jnp.maximum(accv, v)
        accj = jnp.where(better, jp, accj)
    accv_ref[...] = accv
    accj_ref[...] = accj

    @pl.when(i == nsteps - 1)
    def _done():
        accv = accv_ref[...]
        m = jnp.max(accv, axis=1, keepdims=True)
        accj_i = accj_ref[...].astype(jnp.int32)
        jbest = jnp.min(
            jnp.where(accv == m, accj_i, jnp.int32(2**31 - 1)),
            axis=1, keepdims=True)
        row = lax.broadcasted_iota(jnp.int32, (B, 1), 0)
        maxv_ref[...] = m
        idx_ref[...] = jbest - _K1 - row * V


_TBLK = 8192
_TSTEPS = pl.cdiv(S, _TBLK)


def _tail_kernel(bits_ref, logits_ref, mmax_ref, midx_ref, out_ref,
                 accv_ref, accc_ref):
    i = pl.program_id(0)
    neg_inf = jnp.float32(_NEG_INF)

    @pl.when(i == 0)
    def _init():
        accv_ref[...] = jnp.full((B, CHUNK), neg_inf, jnp.float32)
        accc_ref[...] = jnp.zeros((B, CHUNK), jnp.int32)

    chunk_col = lax.broadcasted_iota(jnp.int32, (B, CHUNK), 1)
    accv = accv_ref[...]
    accc = accc_ref[...]
    for k in range(_TBLK // CHUNK):
        off = i * _TBLK + k * CHUNK
        v = _gumbel_from_bits_i32(bits_ref[:, k * CHUNK:(k + 1) * CHUNK]) + \
            logits_ref[:, k * CHUNK:(k + 1) * CHUNK]
        v = jnp.where(chunk_col < S - off, v, neg_inf)
        col = chunk_col + off
        better = v > accv
        accv = jnp.maximum(accv, v)
        accc = jnp.where(better, col, accc)
    accv_ref[...] = accv
    accc_ref[...] = accc

    @pl.when(i == _TSTEPS - 1)
    def _done():
        accv = accv_ref[...]
        m = jnp.max(accv, axis=1, keepdims=True)
        cbest = jnp.min(
            jnp.where(accv == m, accc_ref[...], jnp.int32(2**31 - 1)),
            axis=1, keepdims=True)
        use_tail = m > mmax_ref[...]
        out_ref[...] = jnp.where(use_tail, cbest + VM, midx_ref[...])


@jax.jit
def kernel(logits):
    iota16 = jnp.arange(16, dtype=jnp.int32)
    bits = _sc_bits(iota16).reshape(B, S)

    maxv, midx = pl.pallas_call(
        _main_kernel,
        grid=(VM // BLK,),
        in_specs=[pl.BlockSpec((B, BLK), lambda i: (0, i))],
        out_specs=[pl.BlockSpec((B, 1), lambda i: (0, 0)),
                   pl.BlockSpec((B, 1), lambda i: (0, 0))],
        out_shape=[jax.ShapeDtypeStruct((B, 1), jnp.float32),
                   jax.ShapeDtypeStruct((B, 1), jnp.int32)],
        scratch_shapes=[
            pltpu.VMEM((B, CHUNK), jnp.float32),
            pltpu.VMEM((B, CHUNK), jnp.uint32),
            pltpu.VMEM((B, CHUNK), jnp.uint32),
        ],
    )(logits)

    out = pl.pallas_call(
        _tail_kernel,
        grid=(_TSTEPS,),
        in_specs=[pl.BlockSpec((B, _TBLK), lambda i: (0, i)),
                  pl.BlockSpec((B, _TBLK), lambda i: (0, i + VM // _TBLK)),
                  pl.BlockSpec((B, 1), lambda i: (0, 0)),
                  pl.BlockSpec((B, 1), lambda i: (0, 0))],
        out_specs=pl.BlockSpec((B, 1), lambda i: (0, 0)),
        out_shape=jax.ShapeDtypeStruct((B, 1), jnp.int32),
        scratch_shapes=[
            pltpu.VMEM((B, CHUNK), jnp.float32),
            pltpu.VMEM((B, CHUNK), jnp.int32),
        ],
    )(bits, logits, maxv, midx)
    return out[:, 0].astype(jnp.int64)


# speedup vs baseline: 1.9457x; 1.9457x over previous
import jax
import jax.numpy as jnp
from jax import lax
from jax.experimental import pallas as pl
from jax.experimental.pallas import tpu as pltpu
from jax.experimental.pallas import tpu_sc as plsc

B = 32
V = 1_000_000
BLK = 8192
CHUNK = 1024
VM = 745_472
S = V - VM
SC_NCH = 2
SC_CH = S // SC_NCH
SC_CH_VREGS = SC_CH // 16

_ROTS_EVEN = (13, 15, 26, 6)
_ROTS_ODD = (17, 29, 16, 24)
_K0 = 0
_K1 = 42
_K2 = _K0 ^ _K1 ^ 0x1BD11BDA
_KS = (_K0, _K1, _K2)
_TINY = float(jnp.finfo(jnp.float32).tiny)
_NEG_INF = float("-inf")


def _rotl(x, r):
    return (x << jnp.uint32(r)) + (x >> jnp.uint32(32 - r))


def _rotl_i32(x, r):
    return (x << jnp.int32(r)) + lax.shift_right_logical(x, jnp.int32(32 - r))


def _threefry_bits(x1, rotl, const):
    x0 = x1
    x1 = rotl(x1, _ROTS_EVEN[0]) ^ x0
    for r in _ROTS_EVEN[1:]:
        x0 = x0 + x1
        x1 = rotl(x1, r) ^ x0
    x0 = x0 + const(_KS[1])
    x1 = x1 + const(_KS[2] + 1)
    for g in range(1, 5):
        rots = _ROTS_EVEN if g % 2 == 0 else _ROTS_ODD
        for r in rots:
            x0 = x0 + x1
            x1 = rotl(x1, r) ^ x0
        x0 = x0 + const((_KS[(g + 1) % 3]) & 0xFFFFFFFF)
        x1 = x1 + const((_KS[(g + 2) % 3] + (g + 1)) & 0xFFFFFFFF)
    return x0 ^ x1


def _u32c(v):
    return jnp.uint32(v & 0xFFFFFFFF)


def _i32c(v):
    v &= 0xFFFFFFFF
    return jnp.int32(v - (1 << 32) if v >= (1 << 31) else v)


def _gumbel_from_bits_u32(bits):
    tiny = jnp.float32(_TINY)
    fb = (bits >> jnp.uint32(9)) | jnp.uint32(0x3F800000)
    floats = lax.bitcast_convert_type(fb, jnp.float32) - jnp.float32(1.0)
    u = jnp.maximum(tiny, floats + tiny)
    return -jnp.log(-jnp.log(u))


def _gumbel_from_bits_i32(bits):
    tiny = jnp.float32(_TINY)
    fb = lax.shift_right_logical(bits, jnp.int32(9)) | jnp.int32(0x3F800000)
    floats = lax.bitcast_convert_type(fb, jnp.float32) - jnp.float32(1.0)
    u = jnp.maximum(tiny, floats + tiny)
    return -jnp.log(-jnp.log(u))


def _sc_bits_body(iota_hbm, out_hbm, jv_ref, buf_ref):
    c = lax.axis_index("c")
    s = lax.axis_index("s")
    wid = s * 2 + c
    pltpu.sync_copy(iota_hbm, jv_ref)
    j0 = jv_ref[...] + (wid * V + VM + _K1)

    for ch in range(SC_NCH):
        @pl.loop(0, SC_CH_VREGS, init_carry=j0, unroll=4)
        def _body(t, j):
            bits = _threefry_bits(j, _rotl_i32, _i32c)
            buf_ref[pl.ds(t * 16, 16)] = bits
            return j + 16

        j0 = _body
        pltpu.sync_copy(buf_ref, out_hbm.at[pl.ds(wid * S + ch * SC_CH, SC_CH)])


def _sc_bits(iota16):
    return pl.kernel(
        _sc_bits_body,
        out_type=jax.ShapeDtypeStruct((B * S,), jnp.int32),
        mesh=plsc.VectorSubcoreMesh(core_axis_name="c", subcore_axis_name="s"),
        scratch_types=[
            pltpu.VMEM((16,), jnp.int32),
            pltpu.VMEM((SC_CH,), jnp.int32),
        ],
    )(iota16)


def _main_kernel(logits_ref, maxv_ref, idx_ref, accv_ref, accj_ref, base_ref):
    i = pl.program_id(0)
    nsteps = pl.num_programs(0)
    neg_inf = jnp.float32(_NEG_INF)

    @pl.when(i == 0)
    def _init():
        accv_ref[...] = jnp.full((B, CHUNK), neg_inf, jnp.float32)
        accj_ref[...] = jnp.zeros((B, CHUNK), jnp.uint32)
        row = lax.broadcasted_iota(jnp.int32, (B, CHUNK), 0)
        chunk_col = lax.broadcasted_iota(jnp.int32, (B, CHUNK), 1)
        base_ref[...] = (row * V + chunk_col + _K1).astype(jnp.uint32)

    c0 = i * BLK
    base = base_ref[...]
    accv = accv_ref[...]
    accj = accj_ref[...]
    for k in range(BLK // CHUNK):
        off = c0 + k * CHUNK
        jp = base + off.astype(jnp.uint32)
        v = _gumbel_from_bits_u32(_threefry_bits(jp, _rotl, _u32c)) + \
            logits_ref[:, k * CHUNK:(k + 1) * CHUNK]
        better = v > accv
        accv = jnp.maximum(accv, v)
        accj = jnp.where(better, jp, accj)
    accv_ref[...] = accv
    accj_ref[...] = accj

    @pl.when(i == nsteps - 1)
    def _done():
        accv = accv_ref[...]
        m = jnp.max(accv, axis=1, keepdims=True)
        accj_i = accj_ref[...].astype(jnp.int32)
        jbest = jnp.min(
            jnp.where(accv == m, accj_i, jnp.int32(2**31 - 1)),
            axis=1, keepdims=True)
        row = lax.broadcasted_iota(jnp.int32, (B, 1), 0)
        maxv_ref[...] = m
        idx_ref[...] = jbest - _K1 - row * V


_TBLK = 8192
_TSTEPS = pl.cdiv(S, _TBLK)


def _tail_kernel(bits_ref, logits_ref, mmax_ref, midx_ref, out_ref,
                 accv_ref, accc_ref):
    i = pl.program_id(0)
    neg_inf = jnp.float32(_NEG_INF)

    @pl.when(i == 0)
    def _init():
        accv_ref[...] = jnp.full((B, CHUNK), neg_inf, jnp.float32)
        accc_ref[...] = jnp.zeros((B, CHUNK), jnp.int32)

    chunk_col = lax.broadcasted_iota(jnp.int32, (B, CHUNK), 1)
    accv = accv_ref[...]
    accc = accc_ref[...]
    for k in range(_TBLK // CHUNK):
        off = i * _TBLK + k * CHUNK
        v = _gumbel_from_bits_i32(bits_ref[:, k * CHUNK:(k + 1) * CHUNK]) + \
            logits_ref[:, k * CHUNK:(k + 1) * CHUNK]
        v = jnp.where(chunk_col < S - off, v, neg_inf)
        col = chunk_col + off
        better = v > accv
        accv = jnp.maximum(accv, v)
        accc = jnp.where(better, col, accc)
    accv_ref[...] = accv
    accc_ref[...] = accc

    @pl.when(i == _TSTEPS - 1)
    def _done():
        accv = accv_ref[...]
        m = jnp.max(accv, axis=1, keepdims=True)
        cbest = jnp.min(
            jnp.where(accv == m, accc_ref[...], jnp.int32(2**31 - 1)),
            axis=1, keepdims=True)
        use_tail = m > mmax_ref[...]
        out_ref[...] = jnp.where(use_tail, cbest + VM, midx_ref[...])


@jax.jit
def kernel(logits):
    iota16 = jnp.arange(16, dtype=jnp.int32)
    bits = _sc_bits(iota16).reshape(B, S)

    maxv, midx = pl.pallas_call(
        _main_kernel,
        grid=(VM // BLK,),
        in_specs=[pl.BlockSpec((B, BLK), lambda i: (0, i))],
        out_specs=[pl.BlockSpec((B, 1), lambda i: (0, 0)),
                   pl.BlockSpec((B, 1), lambda i: (0, 0))],
        out_shape=[jax.ShapeDtypeStruct((B, 1), jnp.float32),
                   jax.ShapeDtypeStruct((B, 1), jnp.int32)],
        scratch_shapes=[
            pltpu.VMEM((B, CHUNK), jnp.float32),
            pltpu.VMEM((B, CHUNK), jnp.uint32),
            pltpu.VMEM((B, CHUNK), jnp.uint32),
        ],
    )(logits)

    out = pl.pallas_call(
        _tail_kernel,
        grid=(_TSTEPS,),
        in_specs=[pl.BlockSpec((B, _TBLK), lambda i: (0, i)),
                  pl.BlockSpec((B, _TBLK), lambda i: (0, i + VM // _TBLK)),
                  pl.BlockSpec((B, 1), lambda i: (0, 0)),
                  pl.BlockSpec((B, 1), lambda i: (0, 0))],
        out_specs=pl.BlockSpec((B, 1), lambda i: (0, 0)),
        out_shape=jax.ShapeDtypeStruct((B, 1), jnp.int32),
        scratch_shapes=[
            pltpu.VMEM((B, CHUNK), jnp.float32),
            pltpu.VMEM((B, CHUNK), jnp.int32),
        ],
    )(bits, logits, maxv, midx)
    return out[:, 0].astype(jnp.int64)
